# Initial kernel scaffold; baseline (speedup 1.0000x reference)
import functools

import jax
import jax.numpy as jnp
from jax import lax
from jax.experimental import pallas as pl
from jax.experimental.pallas import tpu as pltpu
from jax.experimental.pallas import tpu_sc as plsc

N = 10000
NPAD = 10240
E = 320000
G = 64
D_IN = 128
D_HID = 512
EPS = 1e-5

NSC = 2
NSUB = 16
NW = NSC * NSUB
EB = 80
EPW = E // NW
NBLK = EPW // EB
ROWS_W = NPAD // NSUB

_HIGH = jax.lax.Precision.HIGHEST


def _dot(a, b):
    return lax.dot_general(a, b, (((1,), (0,)), ((), ())),
                           precision=_HIGH, preferred_element_type=jnp.float32)


def _sc_hist(dst3, batch3, ones16, z16):
    mesh = plsc.VectorSubcoreMesh(core_axis_name="c", subcore_axis_name="s")
    out_type = [
        jax.ShapeDtypeStruct((NSC, NPAD, 16), jnp.float32),
        jax.ShapeDtypeStruct((NSC, 80, 16), jnp.float32),
    ]
    scratch = [
        pltpu.VMEM((NBLK, EB), jnp.int32),
        pltpu.VMEM((4, EB), jnp.int32),
        pltpu.VMEM((EB, 16), jnp.float32),
        pltpu.VMEM_SHARED((NPAD, 16), jnp.float32),
        pltpu.VMEM_SHARED((80, 16), jnp.float32),
    ]

    @functools.partial(pl.kernel, out_type=out_type, mesh=mesh,
                       scratch_types=scratch)
    def k(dst_h, bat_h, ones_h, z_h, degcnt_h, batcnt_h,
          idx_v, bidx_v, ones_v, accd, accb):
        core = lax.axis_index("c")
        sid = lax.axis_index("s")
        w = core * NSUB + sid
        pltpu.sync_copy(ones_h, ones_v)
        pltpu.sync_copy(dst_h.at[w], idx_v)
        pltpu.sync_copy(bat_h.at[w], bidx_v)
        r0 = sid * ROWS_W
        pltpu.sync_copy(z_h.at[pl.ds(r0, ROWS_W)], accd.at[pl.ds(r0, ROWS_W)])

        @pl.when(sid == 0)
        def _():
            pltpu.sync_copy(z_h.at[pl.ds(0, 80)], accb)

        plsc.subcore_barrier()

        @pl.loop(0, NBLK)
        def _(i):
            pltpu.sync_copy(ones_v, accd.at[idx_v.at[i]], add=True)

        @pl.loop(0, 4)
        def _(i):
            pltpu.sync_copy(ones_v, accb.at[bidx_v.at[i]], add=True)

        plsc.subcore_barrier()
        pltpu.sync_copy(accd.at[pl.ds(r0, ROWS_W)],
                        degcnt_h.at[core, pl.ds(r0, ROWS_W)])

        @pl.when(sid == 0)
        def _():
            pltpu.sync_copy(accb, batcnt_h.at[core])

    return k(dst3, batch3, ones16, z16)


def _sc_agg(sources, src3, dst3, z128):
    nchunk = len(sources)
    mesh = plsc.VectorSubcoreMesh(core_axis_name="c", subcore_axis_name="s")
    out_type = [jax.ShapeDtypeStruct((NSC, NPAD, 128), jnp.float32)
                for _ in range(nchunk)]
    scratch = [
        pltpu.VMEM((NBLK, EB), jnp.int32),
        pltpu.VMEM((NBLK, EB), jnp.int32),
        pltpu.VMEM((EB, 128), jnp.float32),
        pltpu.VMEM_SHARED((NPAD, 128), jnp.float32),
    ]

    @functools.partial(pl.kernel, out_type=out_type, mesh=mesh,
                       scratch_types=scratch)
    def k(*refs):
        srcs_h = refs[:nchunk]
        src_h, dst_h, z_h = refs[nchunk:nchunk + 3]
        outs_h = refs[nchunk + 3:nchunk + 3 + nchunk]
        src_v, dst_v, rows_v, acc = refs[nchunk + 3 + nchunk:]
        core = lax.axis_index("c")
        sid = lax.axis_index("s")
        w = core * NSUB + sid
        r0 = sid * ROWS_W
        pltpu.sync_copy(src_h.at[w], src_v)
        pltpu.sync_copy(dst_h.at[w], dst_v)
        for ci in range(nchunk):
            pltpu.sync_copy(z_h.at[pl.ds(r0, ROWS_W)],
                            acc.at[pl.ds(r0, ROWS_W)])
            plsc.subcore_barrier()

            @pl.loop(0, NBLK)
            def _(i):
                pltpu.sync_copy(srcs_h[ci].at[src_v.at[i]], rows_v)
                pltpu.sync_copy(rows_v, acc.at[dst_v.at[i]], add=True)

            plsc.subcore_barrier()
            pltpu.sync_copy(acc.at[pl.ds(r0, ROWS_W)],
                            outs_h[ci].at[core, pl.ds(r0, ROWS_W)])

    out = k(*sources, src3, dst3, z128)
    return out if nchunk > 1 else [out]


NB = 20
RB = NPAD // NB


def _tc_prep(degcnt, x_p):
    def body(dc_ref, x_ref, dinv_ref, xp_ref):
        dc = dc_ref[...]
        deg = dc[0, :, 0:1] + dc[1, :, 0:1] + 1.0
        dinv = lax.rsqrt(deg)
        dinv_ref[...] = dinv
        xp_ref[...] = x_ref[...] * dinv

    return pl.pallas_call(
        body,
        grid=(NB,),
        in_specs=[
            pl.BlockSpec((NSC, RB, 16), lambda p: (0, p, 0)),
            pl.BlockSpec((RB, D_IN), lambda p: (p, 0)),
        ],
        out_specs=[
            pl.BlockSpec((RB, 1), lambda p: (p, 0)),
            pl.BlockSpec((RB, D_IN), lambda p: (p, 0)),
        ],
        out_shape=[
            jax.ShapeDtypeStruct((NPAD, 1), jnp.float32),
            jax.ShapeDtypeStruct((NPAD, D_IN), jnp.float32),
        ],
    )(degcnt, x_p)


def _row_mask(pid):
    rid = pid * RB + lax.broadcasted_iota(jnp.int32, (RB, 1), 0)
    return rid < N


def _stats_update(pid, st_ref, h):
    hm = jnp.where(_row_mask(pid), h, 0.0)

    @pl.when(pid == 0)
    def _():
        st_ref[...] = jnp.zeros_like(st_ref)

    st_ref[0:1, :] += jnp.sum(hm, axis=0, keepdims=True)
    st_ref[1:2, :] += jnp.sum(hm * hm, axis=0, keepdims=True)


def _tc_layer1(agg0, xp, dinv, W1, b1):
    def body(agg_ref, xp_ref, dinv_ref, w_ref, b_ref, h_ref, st_ref):
        pid = pl.program_id(0)
        a = agg_ref[0] + agg_ref[1] + xp_ref[...]
        pre = a * dinv_ref[...]
        h = _dot(pre, w_ref[...]) + b_ref[...]
        h = jnp.maximum(h, 0.0)
        h_ref[...] = h
        _stats_update(pid, st_ref, h)

    return pl.pallas_call(
        body,
        grid=(NB,),
        in_specs=[
            pl.BlockSpec((NSC, RB, D_IN), lambda p: (0, p, 0)),
            pl.BlockSpec((RB, D_IN), lambda p: (p, 0)),
            pl.BlockSpec((RB, 1), lambda p: (p, 0)),
            pl.BlockSpec((D_IN, D_HID), lambda p: (0, 0)),
            pl.BlockSpec((1, D_HID), lambda p: (0, 0)),
        ],
        out_specs=[
            pl.BlockSpec((RB, D_HID), lambda p: (p, 0)),
            pl.BlockSpec((8, D_HID), lambda p: (0, 0)),
        ],
        out_shape=[
            jax.ShapeDtypeStruct((NPAD, D_HID), jnp.float32),
            jax.ShapeDtypeStruct((8, D_HID), jnp.float32),
        ],
    )(agg0, xp, dinv, W1, b1)


def _tc_scale_split(h1, dinv, s1, t1):
    def body(h_ref, dinv_ref, s_ref, t_ref, o0, o1, o2, o3):
        hs = (h_ref[...] * s_ref[...] + t_ref[...]) * dinv_ref[...]
        for ci, o in enumerate((o0, o1, o2, o3)):
            o[...] = hs[:, ci * 128:(ci + 1) * 128]

    return pl.pallas_call(
        body,
        grid=(NB,),
        in_specs=[
            pl.BlockSpec((RB, D_HID), lambda p: (p, 0)),
            pl.BlockSpec((RB, 1), lambda p: (p, 0)),
            pl.BlockSpec((1, D_HID), lambda p: (0, 0)),
            pl.BlockSpec((1, D_HID), lambda p: (0, 0)),
        ],
        out_specs=[pl.BlockSpec((RB, 128), lambda p: (p, 0))] * 4,
        out_shape=[jax.ShapeDtypeStruct((NPAD, 128), jnp.float32)] * 4,
    )(h1, dinv, s1, t1)


def _tc_layer2(agg1, h1, dinv, s1, t1, W2, b2):
    def body(a0_ref, a1_ref, a2_ref, a3_ref, h1_ref, dinv_ref, s_ref, t_ref,
             w_ref, b_ref, h_ref, st_ref):
        pid = pl.program_id(0)
        dinv = dinv_ref[...]
        hself = (h1_ref[...] * s_ref[...] + t_ref[...]) * dinv
        w = w_ref[...]
        acc = jnp.zeros((RB, D_HID), jnp.float32)
        for ci, a_ref in enumerate((a0_ref, a1_ref, a2_ref, a3_ref)):
            a = a_ref[0] + a_ref[1] + hself[:, ci * 128:(ci + 1) * 128]
            acc = acc + _dot(a * dinv, w[ci * 128:(ci + 1) * 128, :])
        h = jnp.maximum(acc + b_ref[...], 0.0)
        h_ref[...] = h
        _stats_update(pid, st_ref, h)

    return pl.pallas_call(
        body,
        grid=(NB,),
        in_specs=(
            [pl.BlockSpec((NSC, RB, 128), lambda p: (0, p, 0))] * 4 + [
                pl.BlockSpec((RB, D_HID), lambda p: (p, 0)),
                pl.BlockSpec((RB, 1), lambda p: (p, 0)),
                pl.BlockSpec((1, D_HID), lambda p: (0, 0)),
                pl.BlockSpec((1, D_HID), lambda p: (0, 0)),
                pl.BlockSpec((D_HID, D_HID), lambda p: (0, 0)),
                pl.BlockSpec((1, D_HID), lambda p: (0, 0)),
            ]),
        out_specs=[
            pl.BlockSpec((RB, D_HID), lambda p: (p, 0)),
            pl.BlockSpec((8, D_HID), lambda p: (0, 0)),
        ],
        out_shape=[
            jax.ShapeDtypeStruct((NPAD, D_HID), jnp.float32),
            jax.ShapeDtypeStruct((8, D_HID), jnp.float32),
        ],
    )(*agg1, h1, dinv, s1, t1, W2, b2)


def _tc_pool(h2, starts, s2, t2):
    def body(starts_ref, h_ref, s_ref, t_ref, out_ref):
        s = s_ref[...]
        t = t_ref[...]
        for g in range(G):
            start = starts_ref[g]
            end = starts_ref[g + 1]
            steps = (end - start + 7) // 8

            def fbody(i, m):
                base = start + i * 8
                rows = h_ref[pl.ds(base, 8), :]
                rid = base + lax.broadcasted_iota(jnp.int32, (8, 1), 0)
                return jnp.maximum(m, jnp.where(rid < end, rows, -jnp.inf))

            m = lax.fori_loop(0, steps,
                              fbody, jnp.full((8, D_HID), -jnp.inf, jnp.float32))
            mx = jnp.max(m, axis=0, keepdims=True)
            out_ref[pl.ds(g, 1), :] = s * mx + t

    return pl.pallas_call(
        body,
        in_specs=[
            pl.BlockSpec(memory_space=pltpu.SMEM),
            pl.BlockSpec((NPAD, D_HID), lambda: (0, 0)),
            pl.BlockSpec((1, D_HID), lambda: (0, 0)),
            pl.BlockSpec((1, D_HID), lambda: (0, 0)),
        ],
        out_specs=pl.BlockSpec((G, D_HID), lambda: (0, 0)),
        out_shape=jax.ShapeDtypeStruct((G, D_HID), jnp.float32),
    )(starts, h2, s2, t2)


def _bn_coeffs(st, gamma, beta):
    mean = st[0] / N
    var = st[1] / N - mean * mean
    s = gamma * lax.rsqrt(var + EPS)
    t = beta - mean * s
    return s.reshape(1, D_HID), t.reshape(1, D_HID)


def kernel(x, edge_index, batch, W1, b1, g1, be1, W2, b2, g2, be2):
    src3 = edge_index[0].reshape(NW, NBLK, EB)
    dst3 = edge_index[1].reshape(NW, NBLK, EB)
    batch3 = jnp.concatenate(
        [batch, jnp.full((NPAD - N,), G, jnp.int32)]).reshape(NW, 4, EB)
    x_p = jnp.pad(x, ((0, NPAD - N), (0, 0)))
    ones16 = jnp.ones((EB, 16), jnp.float32)
    z16 = jnp.zeros((NPAD, 16), jnp.float32)
    z128 = jnp.zeros((NPAD, 128), jnp.float32)

    degcnt, batcnt = _sc_hist(dst3, batch3, ones16, z16)
    dinv, xp = _tc_prep(degcnt, x_p)
    agg0 = _sc_agg([xp], src3, dst3, z128)[0]
    h1, st1 = _tc_layer1(agg0, xp, dinv, W1, b1.reshape(1, D_HID))
    s1, t1 = _bn_coeffs(st1, g1, be1)
    hs_chunks = _tc_scale_split(h1, dinv, s1, t1)
    agg1 = _sc_agg(list(hs_chunks), src3, dst3, z128)
    h2, st2 = _tc_layer2(agg1, h1, dinv, s1, t1, W2, b2.reshape(1, D_HID))
    s2, t2 = _bn_coeffs(st2, g2, be2)

    counts = (batcnt[0, :G, 0] + batcnt[1, :G, 0]).astype(jnp.int32)
    starts = jnp.concatenate(
        [jnp.zeros((1,), jnp.int32), jnp.cumsum(counts)]).astype(jnp.int32)
    return _tc_pool(h2, starts, s2, t2)


# speedup vs baseline: 13.1348x; 13.1348x over previous
import functools

import jax
import jax.numpy as jnp
from jax import lax
from jax.experimental import pallas as pl
from jax.experimental.pallas import tpu as pltpu
from jax.experimental.pallas import tpu_sc as plsc

N = 10000
NPAD = 10240
E = 320000
G = 64
D_IN = 128
D_HID = 512
EPS = 1e-5

NSC = 2
NSUB = 16
NW = NSC * NSUB
EB = 80
EPW = E // NW
NBLK = EPW // EB
ROWS_W = NPAD // NSUB

_HIGH = jax.lax.Precision.HIGHEST


def _dot(a, b):
    return lax.dot_general(a, b, (((1,), (0,)), ((), ())),
                           precision=_HIGH, preferred_element_type=jnp.float32)


def _sc_hist(dst3, batch3, ones128, z128):
    mesh = plsc.VectorSubcoreMesh(core_axis_name="c", subcore_axis_name="s")
    out_type = [
        jax.ShapeDtypeStruct((NSC, NPAD, 128), jnp.float32),
        jax.ShapeDtypeStruct((NSC, 80, 128), jnp.float32),
    ]
    scratch = [
        pltpu.VMEM((NBLK, EB), jnp.int32),
        pltpu.VMEM((4, EB), jnp.int32),
        pltpu.VMEM((EB, 128), jnp.float32),
        pltpu.VMEM_SHARED((NPAD, 128), jnp.float32),
        pltpu.VMEM_SHARED((80, 128), jnp.float32),
    ]

    @functools.partial(pl.kernel, out_type=out_type, mesh=mesh,
                       scratch_types=scratch)
    def k(dst_h, bat_h, ones_h, z_h, degcnt_h, batcnt_h,
          idx_v, bidx_v, ones_v, accd, accb):
        core = lax.axis_index("c")
        sid = lax.axis_index("s")
        w = core * NSUB + sid
        pltpu.sync_copy(ones_h, ones_v)
        pltpu.sync_copy(dst_h.at[w], idx_v)
        pltpu.sync_copy(bat_h.at[w], bidx_v)
        r0 = sid * ROWS_W
        pltpu.sync_copy(z_h.at[pl.ds(r0, ROWS_W)], accd.at[pl.ds(r0, ROWS_W)])

        @pl.when(sid == 0)
        def _():
            pltpu.sync_copy(z_h.at[pl.ds(0, 80)], accb)

        plsc.subcore_barrier()

        @pl.loop(0, NBLK)
        def _(i):
            pltpu.sync_copy(ones_v, accd.at[idx_v.at[i]], add=True)

        @pl.loop(0, 4)
        def _(i):
            pltpu.sync_copy(ones_v, accb.at[bidx_v.at[i]], add=True)

        plsc.subcore_barrier()
        pltpu.sync_copy(accd.at[pl.ds(r0, ROWS_W)],
                        degcnt_h.at[core, pl.ds(r0, ROWS_W)])

        @pl.when(sid == 0)
        def _():
            pltpu.sync_copy(accb, batcnt_h.at[core])

    return k(dst3, batch3, ones128, z128)


def _sc_agg(sources, src3, dst3, z128):
    nchunk = len(sources)
    mesh = plsc.VectorSubcoreMesh(core_axis_name="c", subcore_axis_name="s")
    out_type = [jax.ShapeDtypeStruct((NSC, NPAD, 128), jnp.float32)
                for _ in range(nchunk)]
    scratch = [
        pltpu.VMEM((NBLK, EB), jnp.int32),
        pltpu.VMEM((NBLK, EB), jnp.int32),
        pltpu.VMEM((EB, 128), jnp.float32),
        pltpu.VMEM_SHARED((NPAD, 128), jnp.float32),
    ]

    @functools.partial(pl.kernel, out_type=out_type, mesh=mesh,
                       scratch_types=scratch)
    def k(*refs):
        srcs_h = refs[:nchunk]
        src_h, dst_h, z_h = refs[nchunk:nchunk + 3]
        outs_h = refs[nchunk + 3:nchunk + 3 + nchunk]
        src_v, dst_v, rows_v, acc = refs[nchunk + 3 + nchunk:]
        core = lax.axis_index("c")
        sid = lax.axis_index("s")
        w = core * NSUB + sid
        r0 = sid * ROWS_W
        pltpu.sync_copy(src_h.at[w], src_v)
        pltpu.sync_copy(dst_h.at[w], dst_v)
        for ci in range(nchunk):
            pltpu.sync_copy(z_h.at[pl.ds(r0, ROWS_W)],
                            acc.at[pl.ds(r0, ROWS_W)])
            plsc.subcore_barrier()

            @pl.loop(0, NBLK)
            def _(i):
                pltpu.sync_copy(srcs_h[ci].at[src_v.at[i]], rows_v)
                pltpu.sync_copy(rows_v, acc.at[dst_v.at[i]], add=True)

            plsc.subcore_barrier()
            pltpu.sync_copy(acc.at[pl.ds(r0, ROWS_W)],
                            outs_h[ci].at[core, pl.ds(r0, ROWS_W)])

    out = k(*sources, src3, dst3, z128)
    return list(out) if isinstance(out, (list, tuple)) else [out]


NB = 20
RB = NPAD // NB


def _tc_prep(degcnt, x_p):
    def body(dc_ref, x_ref, dinv_ref, xp_ref):
        dc = dc_ref[...]
        deg = dc[0, :, 0:1] + dc[1, :, 0:1] + 1.0
        dinv = lax.rsqrt(deg)
        dinv_ref[...] = dinv
        xp_ref[...] = x_ref[...] * dinv

    return pl.pallas_call(
        body,
        grid=(NB,),
        in_specs=[
            pl.BlockSpec((NSC, RB, 128), lambda p: (0, p, 0)),
            pl.BlockSpec((RB, D_IN), lambda p: (p, 0)),
        ],
        out_specs=[
            pl.BlockSpec((RB, 1), lambda p: (p, 0)),
            pl.BlockSpec((RB, D_IN), lambda p: (p, 0)),
        ],
        out_shape=[
            jax.ShapeDtypeStruct((NPAD, 1), jnp.float32),
            jax.ShapeDtypeStruct((NPAD, D_IN), jnp.float32),
        ],
    )(degcnt, x_p)


def _row_mask(pid):
    rid = pid * RB + lax.broadcasted_iota(jnp.int32, (RB, 1), 0)
    return rid < N


def _stats_update(pid, st_ref, h):
    hm = jnp.where(_row_mask(pid), h, 0.0)

    @pl.when(pid == 0)
    def _():
        st_ref[...] = jnp.zeros_like(st_ref)

    st_ref[0:1, :] += jnp.sum(hm, axis=0, keepdims=True)
    st_ref[1:2, :] += jnp.sum(hm * hm, axis=0, keepdims=True)


def _tc_layer1(agg0, xp, dinv, W1, b1):
    def body(agg_ref, xp_ref, dinv_ref, w_ref, b_ref, h_ref, st_ref):
        pid = pl.program_id(0)
        a = agg_ref[0] + agg_ref[1] + xp_ref[...]
        pre = a * dinv_ref[...]
        h = _dot(pre, w_ref[...]) + b_ref[...]
        h = jnp.maximum(h, 0.0)
        h_ref[...] = h
        _stats_update(pid, st_ref, h)

    return pl.pallas_call(
        body,
        grid=(NB,),
        in_specs=[
            pl.BlockSpec((NSC, RB, D_IN), lambda p: (0, p, 0)),
            pl.BlockSpec((RB, D_IN), lambda p: (p, 0)),
            pl.BlockSpec((RB, 1), lambda p: (p, 0)),
            pl.BlockSpec((D_IN, D_HID), lambda p: (0, 0)),
            pl.BlockSpec((1, D_HID), lambda p: (0, 0)),
        ],
        out_specs=[
            pl.BlockSpec((RB, D_HID), lambda p: (p, 0)),
            pl.BlockSpec((8, D_HID), lambda p: (0, 0)),
        ],
        out_shape=[
            jax.ShapeDtypeStruct((NPAD, D_HID), jnp.float32),
            jax.ShapeDtypeStruct((8, D_HID), jnp.float32),
        ],
    )(agg0, xp, dinv, W1, b1)


def _tc_scale_split(h1, dinv, s1, t1):
    def body(h_ref, dinv_ref, s_ref, t_ref, o0, o1, o2, o3):
        hs = (h_ref[...] * s_ref[...] + t_ref[...]) * dinv_ref[...]
        for ci, o in enumerate((o0, o1, o2, o3)):
            o[...] = hs[:, ci * 128:(ci + 1) * 128]

    return pl.pallas_call(
        body,
        grid=(NB,),
        in_specs=[
            pl.BlockSpec((RB, D_HID), lambda p: (p, 0)),
            pl.BlockSpec((RB, 1), lambda p: (p, 0)),
            pl.BlockSpec((1, D_HID), lambda p: (0, 0)),
            pl.BlockSpec((1, D_HID), lambda p: (0, 0)),
        ],
        out_specs=[pl.BlockSpec((RB, 128), lambda p: (p, 0))] * 4,
        out_shape=[jax.ShapeDtypeStruct((NPAD, 128), jnp.float32)] * 4,
    )(h1, dinv, s1, t1)


def _tc_layer2(agg1, h1, dinv, s1, t1, W2, b2):
    def body(a0_ref, a1_ref, a2_ref, a3_ref, h1_ref, dinv_ref, s_ref, t_ref,
             w_ref, b_ref, h_ref, st_ref):
        pid = pl.program_id(0)
        dinv = dinv_ref[...]
        hself = (h1_ref[...] * s_ref[...] + t_ref[...]) * dinv
        w = w_ref[...]
        acc = jnp.zeros((RB, D_HID), jnp.float32)
        for ci, a_ref in enumerate((a0_ref, a1_ref, a2_ref, a3_ref)):
            a = a_ref[0] + a_ref[1] + hself[:, ci * 128:(ci + 1) * 128]
            acc = acc + _dot(a * dinv, w[ci * 128:(ci + 1) * 128, :])
        h = jnp.maximum(acc + b_ref[...], 0.0)
        h_ref[...] = h
        _stats_update(pid, st_ref, h)

    return pl.pallas_call(
        body,
        grid=(NB,),
        in_specs=(
            [pl.BlockSpec((NSC, RB, 128), lambda p: (0, p, 0))] * 4 + [
                pl.BlockSpec((RB, D_HID), lambda p: (p, 0)),
                pl.BlockSpec((RB, 1), lambda p: (p, 0)),
                pl.BlockSpec((1, D_HID), lambda p: (0, 0)),
                pl.BlockSpec((1, D_HID), lambda p: (0, 0)),
                pl.BlockSpec((D_HID, D_HID), lambda p: (0, 0)),
                pl.BlockSpec((1, D_HID), lambda p: (0, 0)),
            ]),
        out_specs=[
            pl.BlockSpec((RB, D_HID), lambda p: (p, 0)),
            pl.BlockSpec((8, D_HID), lambda p: (0, 0)),
        ],
        out_shape=[
            jax.ShapeDtypeStruct((NPAD, D_HID), jnp.float32),
            jax.ShapeDtypeStruct((8, D_HID), jnp.float32),
        ],
    )(*agg1, h1, dinv, s1, t1, W2, b2)


def _tc_pool(h2, starts, s2, t2):
    def body(starts_ref, h_ref, s_ref, t_ref, out_ref):
        s = s_ref[...]
        t = t_ref[...]
        for g in range(G):
            start = starts_ref[g]
            end = starts_ref[g + 1]
            base0 = pl.multiple_of((start // 8) * 8, 8)
            steps = (end - base0 + 7) // 8

            def fbody(i, m):
                base = pl.multiple_of(base0 + i * 8, 8)
                rows = h_ref[pl.ds(base, 8), :]
                rid = base + lax.broadcasted_iota(jnp.int32, (8, 1), 0)
                keep = (rid >= start) & (rid < end)
                return jnp.maximum(m, jnp.where(keep, rows, -jnp.inf))

            m = lax.fori_loop(0, steps,
                              fbody, jnp.full((8, D_HID), -jnp.inf, jnp.float32))
            mx = jnp.max(m, axis=0, keepdims=True)
            out_ref[pl.ds(g, 1), :] = s * mx + t

    return pl.pallas_call(
        body,
        in_specs=[
            pl.BlockSpec(memory_space=pltpu.SMEM),
            pl.BlockSpec((NPAD, D_HID), lambda: (0, 0)),
            pl.BlockSpec((1, D_HID), lambda: (0, 0)),
            pl.BlockSpec((1, D_HID), lambda: (0, 0)),
        ],
        out_specs=pl.BlockSpec((G, D_HID), lambda: (0, 0)),
        out_shape=jax.ShapeDtypeStruct((G, D_HID), jnp.float32),
    )(starts, h2, s2, t2)


def _bn_coeffs(st, gamma, beta):
    mean = st[0] / N
    var = st[1] / N - mean * mean
    s = gamma * lax.rsqrt(var + EPS)
    t = beta - mean * s
    return s.reshape(1, D_HID), t.reshape(1, D_HID)


def kernel(x, edge_index, batch, W1, b1, g1, be1, W2, b2, g2, be2):
    src3 = edge_index[0].reshape(NW, NBLK, EB)
    dst3 = edge_index[1].reshape(NW, NBLK, EB)
    batch3 = jnp.concatenate(
        [batch, jnp.full((NPAD - N,), G, jnp.int32)]).reshape(NW, 4, EB)
    x_p = jnp.pad(x, ((0, NPAD - N), (0, 0)))
    ones128 = jnp.ones((EB, 128), jnp.float32)
    z128 = jnp.zeros((NPAD, 128), jnp.float32)

    degcnt, batcnt = _sc_hist(dst3, batch3, ones128, z128)
    dinv, xp = _tc_prep(degcnt, x_p)
    agg0 = _sc_agg([xp], src3, dst3, z128)[0]
    h1, st1 = _tc_layer1(agg0, xp, dinv, W1, b1.reshape(1, D_HID))
    s1, t1 = _bn_coeffs(st1, g1, be1)
    hs_chunks = _tc_scale_split(h1, dinv, s1, t1)
    agg1 = _sc_agg(list(hs_chunks), src3, dst3, z128)
    h2, st2 = _tc_layer2(agg1, h1, dinv, s1, t1, W2, b2.reshape(1, D_HID))
    s2, t2 = _bn_coeffs(st2, g2, be2)

    counts = (batcnt[0, :G, 0] + batcnt[1, :G, 0]).astype(jnp.int32)
    starts = jnp.concatenate(
        [jnp.zeros((1,), jnp.int32), jnp.cumsum(counts)]).astype(jnp.int32)
    return _tc_pool(h2, starts, s2, t2)


# speedup vs baseline: 18.6803x; 1.4222x over previous
import functools

import jax
import jax.numpy as jnp
from jax import lax
from jax.experimental import pallas as pl
from jax.experimental.pallas import tpu as pltpu
from jax.experimental.pallas import tpu_sc as plsc

N = 10000
NPAD = 10240
E = 320000
G = 64
D_IN = 128
D_HID = 512
EPS = 1e-5

NSC = 2
NSUB = 16
NW = NSC * NSUB
EB = 80
EPW = E // NW
NBLK = EPW // EB
GRP = 25
NGRP = NBLK // GRP
BB = NPAD // (NW * EB)
ROWS_W = NPAD // NSUB

_HIGH = jax.lax.Precision.HIGHEST


def _dot(a, b):
    return lax.dot_general(a, b, (((1,), (0,)), ((), ())),
                           precision=_HIGH, preferred_element_type=jnp.float32)


def _sc_hist(dst3, batch3, ones128, z128):
    mesh = plsc.VectorSubcoreMesh(core_axis_name="c", subcore_axis_name="s")
    out_type = [
        jax.ShapeDtypeStruct((NSC, NPAD, 128), jnp.float32),
        jax.ShapeDtypeStruct((NSC, 80, 128), jnp.float32),
    ]
    scratch = [
        pltpu.VMEM((NBLK, EB), jnp.int32),
        pltpu.VMEM((BB, EB), jnp.int32),
        pltpu.VMEM((EB, 128), jnp.float32),
        pltpu.VMEM_SHARED((NPAD, 128), jnp.float32),
        pltpu.VMEM_SHARED((80, 128), jnp.float32),
    ]

    @functools.partial(pl.kernel, out_type=out_type, mesh=mesh,
                       scratch_types=scratch)
    def k(dst_h, bat_h, ones_h, z_h, degcnt_h, batcnt_h,
          idx_v, bidx_v, ones_v, accd, accb):
        core = lax.axis_index("c")
        sid = lax.axis_index("s")
        w = core * NSUB + sid
        pltpu.sync_copy(ones_h, ones_v)
        pltpu.sync_copy(dst_h.at[w], idx_v)
        pltpu.sync_copy(bat_h.at[w], bidx_v)
        r0 = sid * ROWS_W
        pltpu.sync_copy(z_h.at[pl.ds(r0, ROWS_W)], accd.at[pl.ds(r0, ROWS_W)])

        @pl.when(sid == 0)
        def _():
            pltpu.sync_copy(z_h.at[pl.ds(0, 80)], accb)

        plsc.subcore_barrier()

        @pl.loop(0, NBLK)
        def _(i):
            pltpu.sync_copy(ones_v, accd.at[idx_v.at[i]], add=True)

        @pl.loop(0, BB)
        def _(i):
            pltpu.sync_copy(ones_v, accb.at[bidx_v.at[i]], add=True)

        plsc.subcore_barrier()
        pltpu.sync_copy(accd.at[pl.ds(r0, ROWS_W)],
                        degcnt_h.at[core, pl.ds(r0, ROWS_W)])

        @pl.when(sid == 0)
        def _():
            pltpu.sync_copy(accb, batcnt_h.at[core])

    return k(dst3, batch3, ones128, z128)


def _sc_agg(sources, src3, dst3, z128):
    nchunk = len(sources)
    mesh = plsc.VectorSubcoreMesh(core_axis_name="c", subcore_axis_name="s")
    out_type = [jax.ShapeDtypeStruct((NSC, NPAD, 128), jnp.float32)
                for _ in range(nchunk)]
    scratch = [
        pltpu.VMEM((GRP, EB), jnp.int32),
        pltpu.VMEM((GRP, EB), jnp.int32),
        pltpu.VMEM((EB, 128), jnp.float32),
        pltpu.VMEM((EB, 128), jnp.float32),
        pltpu.VMEM_SHARED((NPAD, 128), jnp.float32),
        pltpu.SemaphoreType.DMA,
        pltpu.SemaphoreType.DMA,
    ]

    @functools.partial(pl.kernel, out_type=out_type, mesh=mesh,
                       scratch_types=scratch)
    def k(*refs):
        srcs_h = refs[:nchunk]
        src_h, dst_h, z_h = refs[nchunk:nchunk + 3]
        outs_h = refs[nchunk + 3:nchunk + 3 + nchunk]
        src_v, dst_v, buf0, buf1, acc, sem0, sem1 = refs[nchunk + 3 + nchunk:]
        bufs = (buf0, buf1)
        sems = (sem0, sem1)
        core = lax.axis_index("c")
        sid = lax.axis_index("s")
        w = core * NSUB + sid
        r0 = sid * ROWS_W

        def start(ci, blk, b):
            pltpu.async_copy(srcs_h[ci].at[src_v.at[blk]], bufs[b], sems[b])

        def finish(ci, blk, b):
            pltpu.make_async_copy(srcs_h[ci].at[src_v.at[blk]],
                                  bufs[b], sems[b]).wait()
            pltpu.sync_copy(bufs[b], acc.at[dst_v.at[blk]], add=True)

        for ci in range(nchunk):
            pltpu.sync_copy(z_h.at[pl.ds(r0, ROWS_W)],
                            acc.at[pl.ds(r0, ROWS_W)])
            plsc.subcore_barrier()

            @pl.loop(0, NGRP)
            def _(g):
                pltpu.sync_copy(src_h.at[w, g], src_v)
                pltpu.sync_copy(dst_h.at[w, g], dst_v)
                start(ci, 0, 0)

                @pl.loop(0, (GRP - 1) // 2)
                def _(j):
                    i0 = 2 * j
                    start(ci, i0 + 1, 1)
                    finish(ci, i0, 0)
                    start(ci, i0 + 2, 0)
                    finish(ci, i0 + 1, 1)

                finish(ci, GRP - 1, 0)

            plsc.subcore_barrier()
            pltpu.sync_copy(acc.at[pl.ds(r0, ROWS_W)],
                            outs_h[ci].at[core, pl.ds(r0, ROWS_W)])

    out = k(*sources, src3, dst3, z128)
    return list(out) if isinstance(out, (list, tuple)) else [out]


NB = 20
RB = NPAD // NB


def _tc_prep(degcnt, x_p):
    def body(dc_ref, x_ref, dinv_ref, xp_ref):
        dc = dc_ref[...]
        deg = dc[0, :, 0:1] + dc[1, :, 0:1] + 1.0
        dinv = lax.rsqrt(deg)
        dinv_ref[...] = dinv
        xp_ref[...] = x_ref[...] * dinv

    return pl.pallas_call(
        body,
        grid=(NB,),
        in_specs=[
            pl.BlockSpec((NSC, RB, 128), lambda p: (0, p, 0)),
            pl.BlockSpec((RB, D_IN), lambda p: (p, 0)),
        ],
        out_specs=[
            pl.BlockSpec((RB, 1), lambda p: (p, 0)),
            pl.BlockSpec((RB, D_IN), lambda p: (p, 0)),
        ],
        out_shape=[
            jax.ShapeDtypeStruct((NPAD, 1), jnp.float32),
            jax.ShapeDtypeStruct((NPAD, D_IN), jnp.float32),
        ],
    )(degcnt, x_p)


def _row_mask(pid):
    rid = pid * RB + lax.broadcasted_iota(jnp.int32, (RB, 1), 0)
    return rid < N


def _stats_update(pid, st_ref, h):
    hm = jnp.where(_row_mask(pid), h, 0.0)

    @pl.when(pid == 0)
    def _():
        st_ref[...] = jnp.zeros_like(st_ref)

    st_ref[0:1, :] += jnp.sum(hm, axis=0, keepdims=True)
    st_ref[1:2, :] += jnp.sum(hm * hm, axis=0, keepdims=True)


def _tc_layer1(agg0, xp, dinv, W1, b1):
    def body(agg_ref, xp_ref, dinv_ref, w_ref, b_ref, h_ref, st_ref):
        pid = pl.program_id(0)
        a = agg_ref[0] + agg_ref[1] + xp_ref[...]
        pre = a * dinv_ref[...]
        h = _dot(pre, w_ref[...]) + b_ref[...]
        h = jnp.maximum(h, 0.0)
        h_ref[...] = h
        _stats_update(pid, st_ref, h)

    return pl.pallas_call(
        body,
        grid=(NB,),
        in_specs=[
            pl.BlockSpec((NSC, RB, D_IN), lambda p: (0, p, 0)),
            pl.BlockSpec((RB, D_IN), lambda p: (p, 0)),
            pl.BlockSpec((RB, 1), lambda p: (p, 0)),
            pl.BlockSpec((D_IN, D_HID), lambda p: (0, 0)),
            pl.BlockSpec((1, D_HID), lambda p: (0, 0)),
        ],
        out_specs=[
            pl.BlockSpec((RB, D_HID), lambda p: (p, 0)),
            pl.BlockSpec((8, D_HID), lambda p: (0, 0)),
        ],
        out_shape=[
            jax.ShapeDtypeStruct((NPAD, D_HID), jnp.float32),
            jax.ShapeDtypeStruct((8, D_HID), jnp.float32),
        ],
    )(agg0, xp, dinv, W1, b1)


def _tc_scale_split(h1, dinv, s1, t1):
    def body(h_ref, dinv_ref, s_ref, t_ref, o0, o1, o2, o3):
        hs = (h_ref[...] * s_ref[...] + t_ref[...]) * dinv_ref[...]
        for ci, o in enumerate((o0, o1, o2, o3)):
            o[...] = hs[:, ci * 128:(ci + 1) * 128]

    return pl.pallas_call(
        body,
        grid=(NB,),
        in_specs=[
            pl.BlockSpec((RB, D_HID), lambda p: (p, 0)),
            pl.BlockSpec((RB, 1), lambda p: (p, 0)),
            pl.BlockSpec((1, D_HID), lambda p: (0, 0)),
            pl.BlockSpec((1, D_HID), lambda p: (0, 0)),
        ],
        out_specs=[pl.BlockSpec((RB, 128), lambda p: (p, 0))] * 4,
        out_shape=[jax.ShapeDtypeStruct((NPAD, 128), jnp.float32)] * 4,
    )(h1, dinv, s1, t1)


def _tc_layer2(agg1, h1, dinv, s1, t1, W2, b2):
    def body(a0_ref, a1_ref, a2_ref, a3_ref, h1_ref, dinv_ref, s_ref, t_ref,
             w_ref, b_ref, h_ref, st_ref):
        pid = pl.program_id(0)
        dinv = dinv_ref[...]
        hself = (h1_ref[...] * s_ref[...] + t_ref[...]) * dinv
        w = w_ref[...]
        acc = jnp.zeros((RB, D_HID), jnp.float32)
        for ci, a_ref in enumerate((a0_ref, a1_ref, a2_ref, a3_ref)):
            a = a_ref[0] + a_ref[1] + hself[:, ci * 128:(ci + 1) * 128]
            acc = acc + _dot(a * dinv, w[ci * 128:(ci + 1) * 128, :])
        h = jnp.maximum(acc + b_ref[...], 0.0)
        h_ref[...] = h
        _stats_update(pid, st_ref, h)

    return pl.pallas_call(
        body,
        grid=(NB,),
        in_specs=(
            [pl.BlockSpec((NSC, RB, 128), lambda p: (0, p, 0))] * 4 + [
                pl.BlockSpec((RB, D_HID), lambda p: (p, 0)),
                pl.BlockSpec((RB, 1), lambda p: (p, 0)),
                pl.BlockSpec((1, D_HID), lambda p: (0, 0)),
                pl.BlockSpec((1, D_HID), lambda p: (0, 0)),
                pl.BlockSpec((D_HID, D_HID), lambda p: (0, 0)),
                pl.BlockSpec((1, D_HID), lambda p: (0, 0)),
            ]),
        out_specs=[
            pl.BlockSpec((RB, D_HID), lambda p: (p, 0)),
            pl.BlockSpec((8, D_HID), lambda p: (0, 0)),
        ],
        out_shape=[
            jax.ShapeDtypeStruct((NPAD, D_HID), jnp.float32),
            jax.ShapeDtypeStruct((8, D_HID), jnp.float32),
        ],
    )(*agg1, h1, dinv, s1, t1, W2, b2)


def _tc_pool(h2, starts, s2, t2):
    def body(starts_ref, h_ref, s_ref, t_ref, out_ref):
        s = s_ref[...]
        t = t_ref[...]
        for g in range(G):
            start = starts_ref[g]
            end = starts_ref[g + 1]
            base0 = pl.multiple_of((start // 8) * 8, 8)
            steps = (end - base0 + 7) // 8

            def fbody(i, m):
                base = pl.multiple_of(base0 + i * 8, 8)
                rows = h_ref[pl.ds(base, 8), :]
                rid = base + lax.broadcasted_iota(jnp.int32, (8, 1), 0)
                keep = (rid >= start) & (rid < end)
                return jnp.maximum(m, jnp.where(keep, rows, -jnp.inf))

            m = lax.fori_loop(0, steps,
                              fbody, jnp.full((8, D_HID), -jnp.inf, jnp.float32))
            mx = jnp.max(m, axis=0, keepdims=True)
            out_ref[pl.ds(g, 1), :] = s * mx + t

    return pl.pallas_call(
        body,
        in_specs=[
            pl.BlockSpec(memory_space=pltpu.SMEM),
            pl.BlockSpec((NPAD, D_HID), lambda: (0, 0)),
            pl.BlockSpec((1, D_HID), lambda: (0, 0)),
            pl.BlockSpec((1, D_HID), lambda: (0, 0)),
        ],
        out_specs=pl.BlockSpec((G, D_HID), lambda: (0, 0)),
        out_shape=jax.ShapeDtypeStruct((G, D_HID), jnp.float32),
    )(starts, h2, s2, t2)


def _bn_coeffs(st, gamma, beta):
    mean = st[0] / N
    var = st[1] / N - mean * mean
    s = gamma * lax.rsqrt(var + EPS)
    t = beta - mean * s
    return s.reshape(1, D_HID), t.reshape(1, D_HID)


def kernel(x, edge_index, batch, W1, b1, g1, be1, W2, b2, g2, be2):
    src3 = edge_index[0].reshape(NW, NGRP, GRP, EB)
    dst3 = edge_index[1].reshape(NW, NGRP, GRP, EB)
    dst3h = edge_index[1].reshape(NW, NBLK, EB)
    batch3 = jnp.concatenate(
        [batch, jnp.full((NPAD - N,), G, jnp.int32)]).reshape(NW, BB, EB)
    x_p = jnp.pad(x, ((0, NPAD - N), (0, 0)))
    ones128 = jnp.ones((EB, 128), jnp.float32)
    z128 = jnp.zeros((NPAD, 128), jnp.float32)

    degcnt, batcnt = _sc_hist(dst3h, batch3, ones128, z128)
    dinv, xp = _tc_prep(degcnt, x_p)
    agg0 = _sc_agg([xp], src3, dst3, z128)[0]
    h1, st1 = _tc_layer1(agg0, xp, dinv, W1, b1.reshape(1, D_HID))
    s1, t1 = _bn_coeffs(st1, g1, be1)
    hs_chunks = _tc_scale_split(h1, dinv, s1, t1)
    agg1 = _sc_agg(list(hs_chunks), src3, dst3, z128)
    h2, st2 = _tc_layer2(agg1, h1, dinv, s1, t1, W2, b2.reshape(1, D_HID))
    s2, t2 = _bn_coeffs(st2, g2, be2)

    counts = (batcnt[0, :G, 0] + batcnt[1, :G, 0]).astype(jnp.int32)
    starts = jnp.concatenate(
        [jnp.zeros((1,), jnp.int32), jnp.cumsum(counts)]).astype(jnp.int32)
    return _tc_pool(h2, starts, s2, t2)
